# baseline (speedup 1.0000x reference)
import functools

import jax
import jax.numpy as jnp
from jax import lax
from jax.experimental import pallas as pl
from jax.experimental.pallas import tpu as pltpu
from jax.experimental.pallas import tpu_sc as plsc

_TOKENS = 16384
_D = 256
_CB = 8192
_BT = 512
_BK = 1024
_NT = _TOKENS // _BT
_NK = _CB // _BK

_NC = 2
_NS = 16
_NW = _NC * _NS
_RPW = _TOKENS // _NW
_CH = 128
_NCH = _RPW // _CH


def _dist_argmin_body(x_ref, e_ref, o_ref, bv_ref, bi_ref, p2_ref):
    t = pl.program_id(0)
    k = pl.program_id(1)

    @pl.when(t == 0)
    def _():
        e = e_ref[...]
        p2_ref[k, :] = 0.5 * jnp.sum(e * e, axis=1)

    mm = lax.dot_general(
        x_ref[...].astype(jnp.bfloat16), e_ref[...].astype(jnp.bfloat16),
        (((1,), (1,)), ((), ())), preferred_element_type=jnp.float32)
    s = p2_ref[k, :][None, :] - mm
    lm = jnp.min(s, axis=1, keepdims=True)
    ii = lax.broadcasted_iota(jnp.int32, s.shape, 1) + k * _BK
    li = jnp.min(jnp.where(s == lm, ii, jnp.int32(2**30)), axis=1, keepdims=True)

    @pl.when(k == 0)
    def _():
        bv_ref[...] = lm
        bi_ref[...] = li

    @pl.when(k > 0)
    def _():
        pv = bv_ref[...]
        pi = bi_ref[...]
        upd = lm < pv
        bv_ref[...] = jnp.where(upd, lm, pv)
        bi_ref[...] = jnp.where(upd, li, pi)

    @pl.when(k == _NK - 1)
    def _():
        o_ref[...] = bi_ref[...]


def _dist_argmin(xf, emb, interpret=False):
    return pl.pallas_call(
        _dist_argmin_body,
        grid=(_NT, _NK),
        in_specs=[
            pl.BlockSpec((_BT, _D), lambda t, k: (t, 0)),
            pl.BlockSpec((_BK, _D), lambda t, k: (k, 0)),
        ],
        out_specs=pl.BlockSpec((_BT, 1), lambda t, k: (t, 0)),
        out_shape=jax.ShapeDtypeStruct((_TOKENS, 1), jnp.int32),
        scratch_shapes=[
            pltpu.VMEM((_BT, 1), jnp.float32),
            pltpu.VMEM((_BT, 1), jnp.int32),
            pltpu.VMEM((_NK, _BK), jnp.float32),
        ],
        interpret=interpret,
    )(xf, emb)


def _sc_gather(emb, idx2d):
    mesh = plsc.VectorSubcoreMesh(core_axis_name="c", subcore_axis_name="s")

    @functools.partial(
        pl.kernel,
        mesh=mesh,
        out_type=jax.ShapeDtypeStruct((_TOKENS, _D), jnp.float32),
        scratch_types=[
            pltpu.VMEM((_NCH, _CH), jnp.int32),
            pltpu.VMEM((_CH, _D), jnp.float32),
            pltpu.VMEM((_CH, _D), jnp.float32),
            pltpu.SemaphoreType.DMA,
            pltpu.SemaphoreType.DMA,
        ],
    )
    def gather_k(emb_hbm, idx_hbm, out_hbm, idx_v, rows_a, rows_b, sem_a, sem_b):
        wid = lax.axis_index("s") * _NC + lax.axis_index("c")
        base_chunk = wid * _NCH
        pltpu.sync_copy(idx_hbm.at[pl.ds(base_chunk, _NCH)], idx_v)
        bufs = ((rows_a, sem_a), (rows_b, sem_b))
        copies = [None, None]
        for j in range(_NCH):
            rows, sem = bufs[j % 2]
            copies[j % 2] = pltpu.async_copy(emb_hbm.at[idx_v.at[j]], rows, sem)
            if j >= 1:
                prows, psem = bufs[(j - 1) % 2]
                copies[(j - 1) % 2].wait()
                pltpu.sync_copy(
                    prows, out_hbm.at[pl.ds((base_chunk + j - 1) * _CH, _CH)])
        rows, sem = bufs[(_NCH - 1) % 2]
        copies[(_NCH - 1) % 2].wait()
        pltpu.sync_copy(
            rows, out_hbm.at[pl.ds((base_chunk + _NCH - 1) * _CH, _CH)])

    return gather_k(emb, idx2d)


def kernel(x, emb):
    b = x.shape[0]
    xf = x.reshape(-1, x.shape[-1])
    idx = _dist_argmin(xf, emb)[:, 0]
    q = _sc_gather(emb, idx.reshape(-1, _CH))
    return q.reshape(b, -1, emb.shape[1]), idx.reshape(b, -1)


# speedup vs baseline: 1.2724x; 1.2724x over previous
import functools

import jax
import jax.numpy as jnp
from jax import lax
from jax.experimental import pallas as pl
from jax.experimental.pallas import tpu as pltpu
from jax.experimental.pallas import tpu_sc as plsc

_TOKENS = 16384
_D = 256
_CB = 8192
_BT = 512
_BK = 2048
_NT = _TOKENS // _BT
_NK = _CB // _BK

_NC = 2
_NS = 16
_NW = _NC * _NS
_RPW = _TOKENS // _NW
_CH = 128
_NCH = _RPW // _CH


def _dist_argmin_body(x_ref, e_ref, o_ref, bv_ref, bi_ref, p2_ref):
    t = pl.program_id(0)
    k = pl.program_id(1)

    @pl.when(t == 0)
    def _():
        e = e_ref[...]
        p2_ref[k, :] = 0.5 * jnp.sum(e * e, axis=1)

    mm = lax.dot_general(
        x_ref[...].astype(jnp.bfloat16), e_ref[...].astype(jnp.bfloat16),
        (((1,), (1,)), ((), ())), preferred_element_type=jnp.float32)
    s = p2_ref[k, :][None, :] - mm
    lm = jnp.min(s, axis=1, keepdims=True)
    ii = lax.broadcasted_iota(jnp.int32, s.shape, 1) + k * _BK
    li = jnp.min(jnp.where(s == lm, ii, jnp.int32(2**30)), axis=1, keepdims=True)

    @pl.when(k == 0)
    def _():
        bv_ref[...] = lm
        bi_ref[...] = li

    @pl.when(k > 0)
    def _():
        pv = bv_ref[...]
        pi = bi_ref[...]
        upd = lm < pv
        bv_ref[...] = jnp.where(upd, lm, pv)
        bi_ref[...] = jnp.where(upd, li, pi)

    @pl.when(k == _NK - 1)
    def _():
        o_ref[...] = bi_ref[...]


def _dist_argmin(xf, emb, interpret=False):
    return pl.pallas_call(
        _dist_argmin_body,
        grid=(_NT, _NK),
        in_specs=[
            pl.BlockSpec((_BT, _D), lambda t, k: (t, 0)),
            pl.BlockSpec((_BK, _D), lambda t, k: (k, 0)),
        ],
        out_specs=pl.BlockSpec((_BT, 1), lambda t, k: (t, 0)),
        out_shape=jax.ShapeDtypeStruct((_TOKENS, 1), jnp.int32),
        scratch_shapes=[
            pltpu.VMEM((_BT, 1), jnp.float32),
            pltpu.VMEM((_BT, 1), jnp.int32),
            pltpu.VMEM((_NK, _BK), jnp.float32),
        ],
        interpret=interpret,
    )(xf, emb)


def _sc_gather(emb, idx2d):
    mesh = plsc.VectorSubcoreMesh(core_axis_name="c", subcore_axis_name="s")

    @functools.partial(
        pl.kernel,
        mesh=mesh,
        out_type=jax.ShapeDtypeStruct((_TOKENS, _D), jnp.float32),
        scratch_types=[
            pltpu.VMEM((_NCH, _CH), jnp.int32),
            pltpu.VMEM((_CH, _D), jnp.float32),
            pltpu.VMEM((_CH, _D), jnp.float32),
            pltpu.SemaphoreType.DMA,
            pltpu.SemaphoreType.DMA,
        ],
    )
    def gather_k(emb_hbm, idx_hbm, out_hbm, idx_v, rows_a, rows_b, sem_a, sem_b):
        wid = lax.axis_index("s") * _NC + lax.axis_index("c")
        base_chunk = wid * _NCH
        pltpu.sync_copy(idx_hbm.at[pl.ds(base_chunk, _NCH)], idx_v)
        bufs = ((rows_a, sem_a), (rows_b, sem_b))
        copies = [None, None]
        for j in range(_NCH):
            rows, sem = bufs[j % 2]
            copies[j % 2] = pltpu.async_copy(emb_hbm.at[idx_v.at[j]], rows, sem)
            if j >= 1:
                prows, psem = bufs[(j - 1) % 2]
                copies[(j - 1) % 2].wait()
                pltpu.sync_copy(
                    prows, out_hbm.at[pl.ds((base_chunk + j - 1) * _CH, _CH)])
        rows, sem = bufs[(_NCH - 1) % 2]
        copies[(_NCH - 1) % 2].wait()
        pltpu.sync_copy(
            rows, out_hbm.at[pl.ds((base_chunk + _NCH - 1) * _CH, _CH)])

    return gather_k(emb, idx2d)


def kernel(x, emb):
    b = x.shape[0]
    xf = x.reshape(-1, x.shape[-1])
    idx = _dist_argmin(xf, emb)[:, 0]
    q = _sc_gather(emb, idx.reshape(-1, _CH))
    return q.reshape(b, -1, emb.shape[1]), idx.reshape(b, -1)


# speedup vs baseline: 1.4449x; 1.1355x over previous
import functools

import jax
import jax.numpy as jnp
from jax import lax
from jax.experimental import pallas as pl
from jax.experimental.pallas import tpu as pltpu
from jax.experimental.pallas import tpu_sc as plsc

_TOKENS = 16384
_D = 256
_CB = 8192
_BT = 512
_BK = 4096
_NT = _TOKENS // _BT
_NK = _CB // _BK

_NC = 2
_NS = 16
_NW = _NC * _NS
_RPW = _TOKENS // _NW
_CH = 128
_NCH = _RPW // _CH


def _dist_argmin_body(x_ref, e_ref, o_ref, bv_ref, bi_ref, p2_ref):
    t = pl.program_id(0)
    k = pl.program_id(1)

    @pl.when(t == 0)
    def _():
        e = e_ref[...]
        p2_ref[k, :] = 0.5 * jnp.sum(e * e, axis=1)

    mm = lax.dot_general(
        x_ref[...].astype(jnp.bfloat16), e_ref[...].astype(jnp.bfloat16),
        (((1,), (1,)), ((), ())), preferred_element_type=jnp.float32)
    s = p2_ref[k, :][None, :] - mm
    lm = jnp.min(s, axis=1, keepdims=True)
    ii = lax.broadcasted_iota(jnp.int32, s.shape, 1) + k * _BK
    li = jnp.min(jnp.where(s == lm, ii, jnp.int32(2**30)), axis=1, keepdims=True)

    @pl.when(k == 0)
    def _():
        bv_ref[...] = lm
        bi_ref[...] = li

    @pl.when(k > 0)
    def _():
        pv = bv_ref[...]
        pi = bi_ref[...]
        upd = lm < pv
        bv_ref[...] = jnp.where(upd, lm, pv)
        bi_ref[...] = jnp.where(upd, li, pi)

    @pl.when(k == _NK - 1)
    def _():
        o_ref[...] = bi_ref[...]


def _dist_argmin(xf, emb, interpret=False):
    return pl.pallas_call(
        _dist_argmin_body,
        grid=(_NT, _NK),
        in_specs=[
            pl.BlockSpec((_BT, _D), lambda t, k: (t, 0)),
            pl.BlockSpec((_BK, _D), lambda t, k: (k, 0)),
        ],
        out_specs=pl.BlockSpec((_BT, 1), lambda t, k: (t, 0)),
        out_shape=jax.ShapeDtypeStruct((_TOKENS, 1), jnp.int32),
        scratch_shapes=[
            pltpu.VMEM((_BT, 1), jnp.float32),
            pltpu.VMEM((_BT, 1), jnp.int32),
            pltpu.VMEM((_NK, _BK), jnp.float32),
        ],
        interpret=interpret,
    )(xf, emb)


def _sc_gather(emb, idx2d):
    mesh = plsc.VectorSubcoreMesh(core_axis_name="c", subcore_axis_name="s")

    @functools.partial(
        pl.kernel,
        mesh=mesh,
        out_type=jax.ShapeDtypeStruct((_TOKENS, _D), jnp.float32),
        scratch_types=[
            pltpu.VMEM((_NCH, _CH), jnp.int32),
            pltpu.VMEM((_CH, _D), jnp.float32),
            pltpu.VMEM((_CH, _D), jnp.float32),
            pltpu.SemaphoreType.DMA,
            pltpu.SemaphoreType.DMA,
        ],
    )
    def gather_k(emb_hbm, idx_hbm, out_hbm, idx_v, rows_a, rows_b, sem_a, sem_b):
        wid = lax.axis_index("s") * _NC + lax.axis_index("c")
        base_chunk = wid * _NCH
        pltpu.sync_copy(idx_hbm.at[pl.ds(base_chunk, _NCH)], idx_v)
        bufs = ((rows_a, sem_a), (rows_b, sem_b))
        copies = [None, None]
        for j in range(_NCH):
            rows, sem = bufs[j % 2]
            copies[j % 2] = pltpu.async_copy(emb_hbm.at[idx_v.at[j]], rows, sem)
            if j >= 1:
                prows, psem = bufs[(j - 1) % 2]
                copies[(j - 1) % 2].wait()
                pltpu.sync_copy(
                    prows, out_hbm.at[pl.ds((base_chunk + j - 1) * _CH, _CH)])
        rows, sem = bufs[(_NCH - 1) % 2]
        copies[(_NCH - 1) % 2].wait()
        pltpu.sync_copy(
            rows, out_hbm.at[pl.ds((base_chunk + _NCH - 1) * _CH, _CH)])

    return gather_k(emb, idx2d)


def kernel(x, emb):
    b = x.shape[0]
    xf = x.reshape(-1, x.shape[-1])
    idx = _dist_argmin(xf, emb)[:, 0]
    q = _sc_gather(emb, idx.reshape(-1, _CH))
    return q.reshape(b, -1, emb.shape[1]), idx.reshape(b, -1)


# speedup vs baseline: 1.5578x; 1.0782x over previous
import functools

import jax
import jax.numpy as jnp
from jax import lax
from jax.experimental import pallas as pl
from jax.experimental.pallas import tpu as pltpu
from jax.experimental.pallas import tpu_sc as plsc

_TOKENS = 16384
_D = 256
_CB = 8192
_BT = 512
_BK = 8192
_NT = _TOKENS // _BT
_NK = _CB // _BK

_NC = 2
_NS = 16
_NW = _NC * _NS
_RPW = _TOKENS // _NW
_CH = 128
_NCH = _RPW // _CH


def _dist_argmin_body(x_ref, e_ref, o_ref, bv_ref, bi_ref, p2_ref):
    t = pl.program_id(0)
    k = pl.program_id(1)

    @pl.when(t == 0)
    def _():
        e = e_ref[...]
        p2_ref[k, :] = 0.5 * jnp.sum(e * e, axis=1)

    mm = lax.dot_general(
        x_ref[...].astype(jnp.bfloat16), e_ref[...].astype(jnp.bfloat16),
        (((1,), (1,)), ((), ())), preferred_element_type=jnp.float32)
    s = p2_ref[k, :][None, :] - mm
    lm = jnp.min(s, axis=1, keepdims=True)
    ii = lax.broadcasted_iota(jnp.int32, s.shape, 1) + k * _BK
    li = jnp.min(jnp.where(s == lm, ii, jnp.int32(2**30)), axis=1, keepdims=True)

    @pl.when(k == 0)
    def _():
        bv_ref[...] = lm
        bi_ref[...] = li

    @pl.when(k > 0)
    def _():
        pv = bv_ref[...]
        pi = bi_ref[...]
        upd = lm < pv
        bv_ref[...] = jnp.where(upd, lm, pv)
        bi_ref[...] = jnp.where(upd, li, pi)

    @pl.when(k == _NK - 1)
    def _():
        o_ref[...] = bi_ref[...]


def _dist_argmin(xf, emb, interpret=False):
    return pl.pallas_call(
        _dist_argmin_body,
        grid=(_NT, _NK),
        in_specs=[
            pl.BlockSpec((_BT, _D), lambda t, k: (t, 0)),
            pl.BlockSpec((_BK, _D), lambda t, k: (k, 0)),
        ],
        out_specs=pl.BlockSpec((_BT, 1), lambda t, k: (t, 0)),
        out_shape=jax.ShapeDtypeStruct((_TOKENS, 1), jnp.int32),
        scratch_shapes=[
            pltpu.VMEM((_BT, 1), jnp.float32),
            pltpu.VMEM((_BT, 1), jnp.int32),
            pltpu.VMEM((_NK, _BK), jnp.float32),
        ],
        interpret=interpret,
    )(xf, emb)


def _sc_gather(emb, idx2d):
    mesh = plsc.VectorSubcoreMesh(core_axis_name="c", subcore_axis_name="s")

    @functools.partial(
        pl.kernel,
        mesh=mesh,
        out_type=jax.ShapeDtypeStruct((_TOKENS, _D), jnp.float32),
        scratch_types=[
            pltpu.VMEM((_NCH, _CH), jnp.int32),
            pltpu.VMEM((_CH, _D), jnp.float32),
            pltpu.VMEM((_CH, _D), jnp.float32),
            pltpu.SemaphoreType.DMA,
            pltpu.SemaphoreType.DMA,
        ],
    )
    def gather_k(emb_hbm, idx_hbm, out_hbm, idx_v, rows_a, rows_b, sem_a, sem_b):
        wid = lax.axis_index("s") * _NC + lax.axis_index("c")
        base_chunk = wid * _NCH
        pltpu.sync_copy(idx_hbm.at[pl.ds(base_chunk, _NCH)], idx_v)
        bufs = ((rows_a, sem_a), (rows_b, sem_b))
        copies = [None, None]
        for j in range(_NCH):
            rows, sem = bufs[j % 2]
            copies[j % 2] = pltpu.async_copy(emb_hbm.at[idx_v.at[j]], rows, sem)
            if j >= 1:
                prows, psem = bufs[(j - 1) % 2]
                copies[(j - 1) % 2].wait()
                pltpu.sync_copy(
                    prows, out_hbm.at[pl.ds((base_chunk + j - 1) * _CH, _CH)])
        rows, sem = bufs[(_NCH - 1) % 2]
        copies[(_NCH - 1) % 2].wait()
        pltpu.sync_copy(
            rows, out_hbm.at[pl.ds((base_chunk + _NCH - 1) * _CH, _CH)])

    return gather_k(emb, idx2d)


def kernel(x, emb):
    b = x.shape[0]
    xf = x.reshape(-1, x.shape[-1])
    idx = _dist_argmin(xf, emb)[:, 0]
    q = _sc_gather(emb, idx.reshape(-1, _CH))
    return q.reshape(b, -1, emb.shape[1]), idx.reshape(b, -1)
